# baseline (speedup 1.0000x reference)
import jax
import jax.numpy as jnp
from jax.experimental import pallas as pl
from jax.experimental.pallas import tpu as pltpu

_TN = 2000
_KC = 8


def _knnconv_body(x_ref, w_ref, b_ref, o_ref):
    nk = pl.num_programs(1)
    j = pl.program_id(1)
    tn, kc, d = x_ref.shape
    x = x_ref[...].reshape(tn * kc, d)
    h = jax.lax.dot_general(
        x, w_ref[...],
        dimension_numbers=(((1,), (1,)), ((), ())),
        preferred_element_type=jnp.float32,
    )
    m = jnp.max(h.reshape(tn, kc, h.shape[-1]), axis=1)

    @pl.when(j == 0)
    def _():
        o_ref[...] = m

    @pl.when(j > 0)
    def _():
        o_ref[...] = jnp.maximum(o_ref[...], m)

    @pl.when(j == nk - 1)
    def _():
        o_ref[...] = jnp.maximum(o_ref[...] + b_ref[...], 0.0)


def kernel(agg_feat, W0, b0):
    n, k, d = agg_feat.shape
    o = W0.shape[0]
    grid = (n // _TN, k // _KC)
    b2 = b0.reshape(1, o)
    return pl.pallas_call(
        _knnconv_body,
        grid=grid,
        in_specs=[
            pl.BlockSpec((_TN, _KC, d), lambda i, j: (i, j, 0)),
            pl.BlockSpec((o, d), lambda i, j: (0, 0)),
            pl.BlockSpec((1, o), lambda i, j: (0, 0)),
        ],
        out_specs=pl.BlockSpec((_TN, o), lambda i, j: (i, 0)),
        out_shape=jax.ShapeDtypeStruct((n, o), jnp.float32),
        compiler_params=pltpu.CompilerParams(vmem_limit_bytes=128 * 1024 * 1024),
    )(agg_feat, W0, b2)


# speedup vs baseline: 1.1738x; 1.1738x over previous
import jax
import jax.numpy as jnp
from jax.experimental import pallas as pl


def _knnconv_body(x_ref, w_ref, b_ref, o_ref):
    tn, k, d = x_ref.shape
    x = x_ref[...].reshape(tn * k, d)
    h = jax.lax.dot_general(
        x, w_ref[...],
        dimension_numbers=(((1,), (1,)), ((), ())),
        preferred_element_type=jnp.float32,
    )
    h = h.reshape(tn, k, h.shape[-1])
    pooled = jnp.max(h, axis=1) + b_ref[...]
    o_ref[...] = jnp.maximum(pooled, 0.0)


def kernel(agg_feat, W0, b0):
    n, k, d = agg_feat.shape
    o = W0.shape[0]
    tn = 1000
    grid = n // tn
    b2 = b0.reshape(1, o)
    return pl.pallas_call(
        _knnconv_body,
        grid=(grid,),
        in_specs=[
            pl.BlockSpec((tn, k, d), lambda i: (i, 0, 0)),
            pl.BlockSpec((o, d), lambda i: (0, 0)),
            pl.BlockSpec((1, o), lambda i: (0, 0)),
        ],
        out_specs=pl.BlockSpec((tn, o), lambda i: (i, 0)),
        out_shape=jax.ShapeDtypeStruct((n, o), jnp.float32),
    )(agg_feat, W0, b2)
